# baseline (speedup 1.0000x reference)
import functools

import jax
import jax.numpy as jnp
from jax import lax
from jax.experimental import pallas as pl
from jax.experimental.pallas import tpu as pltpu
from jax.experimental.pallas import tpu_sc as plsc

BATCH = 16
SEQ_LEN = 1024
TOTAL = BATCH * SEQ_LEN
EMB_DIM = 64
ENC_SIZE = 512
REL_DIM = 256
PAIR = 2 * EMB_DIM

NUM_WORKERS = 32
TPW = TOTAL // NUM_WORKERS
CHUNK = 128
NCHUNK = TPW // CHUNK


def _gather_body(ids_pos, ids_cat, ids_sense, flat_idx,
                 pos_t, cat_t, sense_t, head_t,
                 out_pos, out_cat, out_sense, out_head,
                 idx_v, p_v, c_v, s_v, h_v, gsem, wsem):
    wid = lax.axis_index("s") * 2 + lax.axis_index("c")
    base = wid * TPW
    staged = []
    for k, ids in enumerate((ids_pos, ids_cat, ids_sense, flat_idx)):
        for j in range(NCHUNK):
            rows = pl.ds(base + j * CHUNK, CHUNK)
            staged.append(
                pltpu.async_copy(ids.at[rows], idx_v.at[k * NCHUNK + j], gsem))
    for h in staged:
        h.wait()
    for j in range(NCHUNK):
        rows = pl.ds(base + j * CHUNK, CHUNK)
        gathers = (
            pltpu.async_copy(pos_t.at[idx_v.at[0 * NCHUNK + j]], p_v, gsem),
            pltpu.async_copy(cat_t.at[idx_v.at[1 * NCHUNK + j]], c_v, gsem),
            pltpu.async_copy(sense_t.at[idx_v.at[2 * NCHUNK + j]], s_v, gsem),
            pltpu.async_copy(head_t.at[idx_v.at[3 * NCHUNK + j]], h_v, gsem),
        )
        for h in gathers:
            h.wait()
        writes = (
            pltpu.async_copy(p_v, out_pos.at[rows], wsem),
            pltpu.async_copy(c_v, out_cat.at[rows], wsem),
            pltpu.async_copy(s_v, out_sense.at[rows], wsem),
            pltpu.async_copy(h_v, out_head.at[rows], wsem),
        )
        for h in writes:
            h.wait()


_gather = functools.partial(
    pl.kernel,
    mesh=plsc.VectorSubcoreMesh(core_axis_name="c", subcore_axis_name="s"),
    out_type=(
        jax.ShapeDtypeStruct((TOTAL, PAIR), jnp.float32),
        jax.ShapeDtypeStruct((TOTAL, PAIR), jnp.float32),
        jax.ShapeDtypeStruct((TOTAL, PAIR), jnp.float32),
        jax.ShapeDtypeStruct((TOTAL, REL_DIM), jnp.float32),
    ),
    scratch_types=[
        pltpu.VMEM((16, CHUNK), jnp.int32),
        pltpu.VMEM((CHUNK, PAIR), jnp.float32),
        pltpu.VMEM((CHUNK, PAIR), jnp.float32),
        pltpu.VMEM((CHUNK, PAIR), jnp.float32),
        pltpu.VMEM((CHUNK, REL_DIM), jnp.float32),
        pltpu.SemaphoreType.DMA,
        pltpu.SemaphoreType.DMA,
    ],
)(_gather_body)


CAT_VOCAB = 100000
POS_VOCAB = 1000
PAIR_BLK = 10000


def _pair_rows(x):
    nxt = jnp.concatenate([x[1:], x[:1]], axis=0)
    return jnp.concatenate([x, nxt], axis=1)


def _pairize_body(cat_ref, sense_ref, pos_ref, oc_ref, os_ref, op_ref):
    oc_ref[...] = _pair_rows(cat_ref[...])
    os_ref[...] = _pair_rows(sense_ref[...])

    @pl.when(pl.program_id(0) == 0)
    def _():
        op_ref[...] = _pair_rows(pos_ref[...])


def _pairize(cat_t, sense_t, pos_t):
    return pl.pallas_call(
        _pairize_body,
        grid=(CAT_VOCAB // PAIR_BLK,),
        in_specs=[
            pl.BlockSpec((PAIR_BLK, EMB_DIM), lambda i: (i, 0)),
            pl.BlockSpec((PAIR_BLK, EMB_DIM), lambda i: (i, 0)),
            pl.BlockSpec((POS_VOCAB, EMB_DIM), lambda i: (0, 0)),
        ],
        out_specs=[
            pl.BlockSpec((PAIR_BLK, PAIR), lambda i: (i, 0)),
            pl.BlockSpec((PAIR_BLK, PAIR), lambda i: (i, 0)),
            pl.BlockSpec((POS_VOCAB, PAIR), lambda i: (0, 0)),
        ],
        out_shape=[
            jax.ShapeDtypeStruct((CAT_VOCAB, PAIR), jnp.float32),
            jax.ShapeDtypeStruct((CAT_VOCAB, PAIR), jnp.float32),
            jax.ShapeDtypeStruct((POS_VOCAB, PAIR), jnp.float32),
        ],
    )(cat_t, sense_t, pos_t)


BM = 1024


def _head_body(x_ref, w_ref, o_ref):
    o_ref[...] = jnp.dot(x_ref[...], w_ref[...],
                         preferred_element_type=jnp.float32)


def _head_proj(x, wh):
    return pl.pallas_call(
        _head_body,
        grid=(TOTAL // BM,),
        in_specs=[
            pl.BlockSpec((BM, ENC_SIZE), lambda i: (i, 0)),
            pl.BlockSpec((ENC_SIZE, REL_DIM), lambda i: (0, 0)),
        ],
        out_specs=pl.BlockSpec((BM, REL_DIM), lambda i: (i, 0)),
        out_shape=jax.ShapeDtypeStruct((TOTAL, REL_DIM), jnp.float32),
    )(x, wh)


def _half(g, par):
    lo = g[:, :EMB_DIM]
    hi = g[:, EMB_DIM:]
    return jnp.where(par == 1, hi, lo)


def _mm_body(gp_ref, gc_ref, gs_ref, gh_ref, ids_ref,
             wp_ref, wc_ref, ws_ref, b_ref, o_ref):
    ids = ids_ref[...]
    acc = gh_ref[...] + b_ref[...]
    acc += jnp.dot(_half(gp_ref[...], ids[:, 0:1] & 1), wp_ref[...],
                   preferred_element_type=jnp.float32)
    acc += jnp.dot(_half(gc_ref[...], ids[:, 1:2] & 1), wc_ref[...],
                   preferred_element_type=jnp.float32)
    acc += jnp.dot(_half(gs_ref[...], ids[:, 2:3] & 1), ws_ref[...],
                   preferred_element_type=jnp.float32)
    o_ref[...] = jnp.maximum(acc, 0.0)


def _matmul(gp, gc, gs, gh, ids3, wp, wc, ws, b2d):
    pair_spec = pl.BlockSpec((BM, PAIR), lambda i: (i, 0))
    w_spec = pl.BlockSpec((EMB_DIM, REL_DIM), lambda i: (0, 0))
    return pl.pallas_call(
        _mm_body,
        grid=(TOTAL // BM,),
        in_specs=[
            pair_spec, pair_spec, pair_spec,
            pl.BlockSpec((BM, REL_DIM), lambda i: (i, 0)),
            pl.BlockSpec((BM, 3), lambda i: (i, 0)),
            w_spec, w_spec, w_spec,
            pl.BlockSpec((1, REL_DIM), lambda i: (0, 0)),
        ],
        out_specs=pl.BlockSpec((BM, REL_DIM), lambda i: (i, 0)),
        out_shape=jax.ShapeDtypeStruct((TOTAL, REL_DIM), jnp.float32),
    )(gp, gc, gs, gh, ids3, wp, wc, ws, b2d)


def kernel(input_data, index, src_enc_data, pos_table, cat_table, sense_table,
           W, b, lengths):
    ids_pos = input_data[:, 0].astype(jnp.int32)
    ids_cat = input_data[:, 1].astype(jnp.int32)
    ids_sense = input_data[:, 2].astype(jnp.int32)
    t = jnp.arange(TOTAL, dtype=jnp.int32)
    flat_idx = (t // SEQ_LEN) * SEQ_LEN + index.astype(jnp.int32)
    cat2, sense2, pos2 = _pairize(cat_table, sense_table, pos_table)
    wp = W[:EMB_DIM]
    wc = W[EMB_DIM:2 * EMB_DIM]
    ws = W[2 * EMB_DIM:3 * EMB_DIM]
    wh = W[3 * EMB_DIM:]
    hproj = _head_proj(src_enc_data, wh)
    gp, gc, gs, gh = _gather(ids_pos & ~1, ids_cat & ~1, ids_sense & ~1,
                             flat_idx, pos2, cat2, sense2, hproj)
    return _matmul(gp, gc, gs, gh, input_data.astype(jnp.int32),
                   wp, wc, ws, b.reshape(1, REL_DIM))


# speedup vs baseline: 1.1338x; 1.1338x over previous
import functools

import jax
import jax.numpy as jnp
from jax import lax
from jax.experimental import pallas as pl
from jax.experimental.pallas import tpu as pltpu
from jax.experimental.pallas import tpu_sc as plsc

BATCH = 16
SEQ_LEN = 1024
TOTAL = BATCH * SEQ_LEN
HALF = TOTAL // 2
EMB_DIM = 64
ENC_SIZE = 512
REL_DIM = 256
PAIR = 2 * EMB_DIM

NUM_WORKERS = 32
TPW = TOTAL // NUM_WORKERS
CHUNK = 128
ECH = CHUNK // 2
NCHUNK = TPW // CHUNK


def _emb_body(ipe, ipo, ice, ico, ise, iso,
              pos_t, cat_t, sense_t,
              out_pos, out_cat, out_sense,
              idx_v, pe_v, po_v, ce_v, co_v, se_v, so_v, gsem, wsem):
    wid = lax.axis_index("s") * 2 + lax.axis_index("c")
    pbase = wid * (TPW // 2)
    streams = ((ipe, ipo), (ice, ico), (ise, iso))
    staged = []
    for k, (ide, ido) in enumerate(streams):
        for j in range(NCHUNK):
            prows = pl.ds(pbase + j * ECH, ECH)
            staged.append(pltpu.async_copy(
                ide.at[prows], idx_v.at[(2 * k) * NCHUNK + j], gsem))
            staged.append(pltpu.async_copy(
                ido.at[prows], idx_v.at[(2 * k + 1) * NCHUNK + j], gsem))
    for h in staged:
        h.wait()
    bufs = ((pe_v, po_v), (ce_v, co_v), (se_v, so_v))
    tables = (pos_t, cat_t, sense_t)
    outs = (out_pos, out_cat, out_sense)
    for j in range(NCHUNK):
        prows = pl.ds(pbase + j * ECH, ECH)
        gathers = []
        for k in range(3):
            gathers.append(pltpu.async_copy(
                tables[k].at[idx_v.at[(2 * k) * NCHUNK + j]], bufs[k][0], gsem))
            gathers.append(pltpu.async_copy(
                tables[k].at[idx_v.at[(2 * k + 1) * NCHUNK + j]], bufs[k][1],
                gsem))
        for h in gathers:
            h.wait()
        writes = []
        for k in range(3):
            writes.append(pltpu.async_copy(
                bufs[k][0], outs[k].at[prows, pl.ds(0, EMB_DIM)], wsem))
            writes.append(pltpu.async_copy(
                bufs[k][1], outs[k].at[prows, pl.ds(EMB_DIM, EMB_DIM)], wsem))
        for h in writes:
            h.wait()


_emb_gather = functools.partial(
    pl.kernel,
    mesh=plsc.VectorSubcoreMesh(core_axis_name="c", subcore_axis_name="s"),
    out_type=(
        jax.ShapeDtypeStruct((HALF, PAIR), jnp.float32),
        jax.ShapeDtypeStruct((HALF, PAIR), jnp.float32),
        jax.ShapeDtypeStruct((HALF, PAIR), jnp.float32),
    ),
    scratch_types=[
        pltpu.VMEM((24, ECH), jnp.int32),
        pltpu.VMEM((ECH, EMB_DIM), jnp.float32),
        pltpu.VMEM((ECH, EMB_DIM), jnp.float32),
        pltpu.VMEM((ECH, EMB_DIM), jnp.float32),
        pltpu.VMEM((ECH, EMB_DIM), jnp.float32),
        pltpu.VMEM((ECH, EMB_DIM), jnp.float32),
        pltpu.VMEM((ECH, EMB_DIM), jnp.float32),
        pltpu.SemaphoreType.DMA,
        pltpu.SemaphoreType.DMA,
    ],
    compiler_params=pltpu.CompilerParams(use_tc_tiling_on_sc=False),
)(_emb_body)


def _head_gather_body(ihe, iho, head_t, out_head,
                      idx_v, he_v, ho_v, gsem, wsem):
    wid = lax.axis_index("s") * 2 + lax.axis_index("c")
    pbase = wid * (TPW // 2)
    staged = []
    for j in range(NCHUNK):
        prows = pl.ds(pbase + j * ECH, ECH)
        staged.append(pltpu.async_copy(ihe.at[prows], idx_v.at[2 * j], gsem))
        staged.append(pltpu.async_copy(iho.at[prows], idx_v.at[2 * j + 1],
                                       gsem))
    for h in staged:
        h.wait()
    for j in range(NCHUNK):
        prows = pl.ds(pbase + j * ECH, ECH)
        g = (pltpu.async_copy(head_t.at[idx_v.at[2 * j]], he_v, gsem),
             pltpu.async_copy(head_t.at[idx_v.at[2 * j + 1]], ho_v, gsem))
        for h in g:
            h.wait()
        w = (pltpu.async_copy(he_v, out_head.at[prows, pl.ds(0, REL_DIM)],
                              wsem),
             pltpu.async_copy(ho_v, out_head.at[prows, pl.ds(REL_DIM,
                                                             REL_DIM)], wsem))
        for h in w:
            h.wait()


_head_gather = functools.partial(
    pl.kernel,
    mesh=plsc.VectorSubcoreMesh(core_axis_name="c", subcore_axis_name="s"),
    out_type=jax.ShapeDtypeStruct((HALF, 2 * REL_DIM), jnp.float32),
    scratch_types=[
        pltpu.VMEM((8, ECH), jnp.int32),
        pltpu.VMEM((ECH, REL_DIM), jnp.float32),
        pltpu.VMEM((ECH, REL_DIM), jnp.float32),
        pltpu.SemaphoreType.DMA,
        pltpu.SemaphoreType.DMA,
    ],
)(_head_gather_body)


BM = 1024


def _head_body(x_ref, w_ref, o_ref):
    o_ref[...] = jnp.dot(x_ref[...], w_ref[...],
                         preferred_element_type=jnp.float32)


def _head_proj(x, wh):
    return pl.pallas_call(
        _head_body,
        grid=(TOTAL // BM,),
        in_specs=[
            pl.BlockSpec((BM, ENC_SIZE), lambda i: (i, 0)),
            pl.BlockSpec((ENC_SIZE, REL_DIM), lambda i: (0, 0)),
        ],
        out_specs=pl.BlockSpec((BM, REL_DIM), lambda i: (i, 0)),
        out_shape=jax.ShapeDtypeStruct((TOTAL, REL_DIM), jnp.float32),
    )(x, wh)


BM2 = BM // 2


def _mm_body(gp_ref, gc_ref, gs_ref, gh_ref, wp_ref, wc_ref, ws_ref,
             b_ref, o_ref):
    acc = gh_ref[...] + b_ref[...]
    acc += jnp.dot(gp_ref[...], wp_ref[...],
                   preferred_element_type=jnp.float32)
    acc += jnp.dot(gc_ref[...], wc_ref[...],
                   preferred_element_type=jnp.float32)
    acc += jnp.dot(gs_ref[...], ws_ref[...],
                   preferred_element_type=jnp.float32)
    o_ref[...] = jnp.maximum(acc, 0.0)


def _matmul(gp, gc, gs, gh, wp2, wc2, ws2, b2):
    pair_spec = pl.BlockSpec((BM2, PAIR), lambda i: (i, 0))
    w_spec = pl.BlockSpec((PAIR, 2 * REL_DIM), lambda i: (0, 0))
    return pl.pallas_call(
        _mm_body,
        grid=(HALF // BM2,),
        in_specs=[
            pair_spec, pair_spec, pair_spec,
            pl.BlockSpec((BM2, 2 * REL_DIM), lambda i: (i, 0)),
            w_spec, w_spec, w_spec,
            pl.BlockSpec((1, 2 * REL_DIM), lambda i: (0, 0)),
        ],
        out_specs=pl.BlockSpec((BM2, 2 * REL_DIM), lambda i: (i, 0)),
        out_shape=jax.ShapeDtypeStruct((HALF, 2 * REL_DIM), jnp.float32),
    )(gp, gc, gs, gh, wp2, wc2, ws2, b2)


def _blockdiag(w):
    z = jnp.zeros((EMB_DIM, REL_DIM), jnp.float32)
    return jnp.concatenate(
        [jnp.concatenate([w, z], axis=1),
         jnp.concatenate([z, w], axis=1)], axis=0)


def kernel(input_data, index, src_enc_data, pos_table, cat_table, sense_table,
           W, b, lengths):
    ids = input_data.astype(jnp.int32)
    t = jnp.arange(TOTAL, dtype=jnp.int32)
    flat_idx = (t // SEQ_LEN) * SEQ_LEN + index.astype(jnp.int32)
    ipe, ice, ise = ids[0::2, 0], ids[0::2, 1], ids[0::2, 2]
    ipo, ico, iso = ids[1::2, 0], ids[1::2, 1], ids[1::2, 2]
    ihe, iho = flat_idx[0::2], flat_idx[1::2]
    wp = W[:EMB_DIM]
    wc = W[EMB_DIM:2 * EMB_DIM]
    ws = W[2 * EMB_DIM:3 * EMB_DIM]
    wh = W[3 * EMB_DIM:]
    hproj = _head_proj(src_enc_data, wh)
    gp, gc, gs = _emb_gather(ipe, ipo, ice, ico, ise, iso,
                             pos_table, cat_table, sense_table)
    gh = _head_gather(ihe, iho, hproj)
    out2 = _matmul(gp, gc, gs, gh, _blockdiag(wp), _blockdiag(wc),
                   _blockdiag(ws),
                   jnp.concatenate([b, b]).reshape(1, 2 * REL_DIM))
    return out2.reshape(TOTAL, REL_DIM)


# speedup vs baseline: 1.2249x; 1.0804x over previous
import functools

import jax
import jax.numpy as jnp
from jax import lax
from jax.experimental import pallas as pl
from jax.experimental.pallas import tpu as pltpu
from jax.experimental.pallas import tpu_sc as plsc

BATCH = 16
SEQ_LEN = 1024
TOTAL = BATCH * SEQ_LEN
HALF = TOTAL // 2
EMB_DIM = 64
ENC_SIZE = 512
REL_DIM = 256
PAIR = 2 * EMB_DIM

NUM_WORKERS = 32
TPW = TOTAL // NUM_WORKERS
CHUNK = 128
ECH = CHUNK // 2
NCHUNK = TPW // CHUNK


def _emb_body(ipe, ipo, ice, ico, ise, iso,
              pos_t, cat_t, sense_t,
              out_pos, out_cat, out_sense,
              idx_v, pe_v, po_v, ce_v, co_v, se_v, so_v, gsem, wsem):
    wid = lax.axis_index("s") * 2 + lax.axis_index("c")
    pbase = wid * (TPW // 2)
    streams = ((ipe, ipo), (ice, ico), (ise, iso))
    staged = []
    for k, (ide, ido) in enumerate(streams):
        for j in range(NCHUNK):
            prows = pl.ds(pbase + j * ECH, ECH)
            staged.append(pltpu.async_copy(
                ide.at[prows], idx_v.at[(2 * k) * NCHUNK + j], gsem))
            staged.append(pltpu.async_copy(
                ido.at[prows], idx_v.at[(2 * k + 1) * NCHUNK + j], gsem))
    for h in staged:
        h.wait()
    bufs = ((pe_v, po_v), (ce_v, co_v), (se_v, so_v))
    tables = (pos_t, cat_t, sense_t)
    outs = (out_pos, out_cat, out_sense)
    for j in range(NCHUNK):
        prows = pl.ds(pbase + j * ECH, ECH)
        gathers = []
        for k in range(3):
            gathers.append(pltpu.async_copy(
                tables[k].at[idx_v.at[(2 * k) * NCHUNK + j]], bufs[k][0], gsem))
            gathers.append(pltpu.async_copy(
                tables[k].at[idx_v.at[(2 * k + 1) * NCHUNK + j]], bufs[k][1],
                gsem))
        for h in gathers:
            h.wait()
        writes = []
        for k in range(3):
            writes.append(pltpu.async_copy(
                bufs[k][0], outs[k].at[prows, pl.ds(0, EMB_DIM)], wsem))
            writes.append(pltpu.async_copy(
                bufs[k][1], outs[k].at[prows, pl.ds(EMB_DIM, EMB_DIM)], wsem))
        for h in writes:
            h.wait()


_emb_gather = functools.partial(
    pl.kernel,
    mesh=plsc.VectorSubcoreMesh(core_axis_name="c", subcore_axis_name="s"),
    out_type=(
        jax.ShapeDtypeStruct((HALF, PAIR), jnp.float32),
        jax.ShapeDtypeStruct((HALF, PAIR), jnp.float32),
        jax.ShapeDtypeStruct((HALF, PAIR), jnp.float32),
    ),
    scratch_types=[
        pltpu.VMEM((24, ECH), jnp.int32),
        pltpu.VMEM((ECH, EMB_DIM), jnp.float32),
        pltpu.VMEM((ECH, EMB_DIM), jnp.float32),
        pltpu.VMEM((ECH, EMB_DIM), jnp.float32),
        pltpu.VMEM((ECH, EMB_DIM), jnp.float32),
        pltpu.VMEM((ECH, EMB_DIM), jnp.float32),
        pltpu.VMEM((ECH, EMB_DIM), jnp.float32),
        pltpu.SemaphoreType.DMA,
        pltpu.SemaphoreType.DMA,
    ],
    compiler_params=pltpu.CompilerParams(use_tc_tiling_on_sc=False),
)(_emb_body)


def _head_gather_body(ihe, iho, head_t, out_head,
                      idx_v, he_v, ho_v, gsem, wsem):
    wid = lax.axis_index("s") * 2 + lax.axis_index("c")
    pbase = wid * (TPW // 2)
    staged = []
    for j in range(NCHUNK):
        prows = pl.ds(pbase + j * ECH, ECH)
        staged.append(pltpu.async_copy(ihe.at[prows], idx_v.at[2 * j], gsem))
        staged.append(pltpu.async_copy(iho.at[prows], idx_v.at[2 * j + 1],
                                       gsem))
    for h in staged:
        h.wait()
    for j in range(NCHUNK):
        prows = pl.ds(pbase + j * ECH, ECH)
        g = (pltpu.async_copy(head_t.at[idx_v.at[2 * j]], he_v, gsem),
             pltpu.async_copy(head_t.at[idx_v.at[2 * j + 1]], ho_v, gsem))
        for h in g:
            h.wait()
        w = (pltpu.async_copy(he_v, out_head.at[prows, pl.ds(0, REL_DIM)],
                              wsem),
             pltpu.async_copy(ho_v, out_head.at[prows, pl.ds(REL_DIM,
                                                             REL_DIM)], wsem))
        for h in w:
            h.wait()


_head_gather = functools.partial(
    pl.kernel,
    mesh=plsc.VectorSubcoreMesh(core_axis_name="c", subcore_axis_name="s"),
    out_type=jax.ShapeDtypeStruct((HALF, 2 * REL_DIM), jnp.float32),
    scratch_types=[
        pltpu.VMEM((8, ECH), jnp.int32),
        pltpu.VMEM((ECH, REL_DIM), jnp.float32),
        pltpu.VMEM((ECH, REL_DIM), jnp.float32),
        pltpu.SemaphoreType.DMA,
        pltpu.SemaphoreType.DMA,
    ],
)(_head_gather_body)


BM = 1024


def _head_body(x_ref, w_ref, o_ref):
    o_ref[...] = jnp.dot(x_ref[...], w_ref[...],
                         preferred_element_type=jnp.float32)


def _head_proj(x, wh):
    return pl.pallas_call(
        _head_body,
        grid=(TOTAL // BM,),
        in_specs=[
            pl.BlockSpec((BM, ENC_SIZE), lambda i: (i, 0)),
            pl.BlockSpec((ENC_SIZE, REL_DIM), lambda i: (0, 0)),
        ],
        out_specs=pl.BlockSpec((BM, REL_DIM), lambda i: (i, 0)),
        out_shape=jax.ShapeDtypeStruct((TOTAL, REL_DIM), jnp.float32),
    )(x, wh)


BM2 = BM // 2


def _mm_body(gp_ref, gc_ref, gs_ref, gh_ref, wp_ref, wc_ref, ws_ref,
             b_ref, o_ref):
    acc = gh_ref[...] + b_ref[...]
    acc += jnp.dot(gp_ref[...], wp_ref[...],
                   preferred_element_type=jnp.float32)
    acc += jnp.dot(gc_ref[...], wc_ref[...],
                   preferred_element_type=jnp.float32)
    acc += jnp.dot(gs_ref[...], ws_ref[...],
                   preferred_element_type=jnp.float32)
    o_ref[...] = jnp.maximum(acc, 0.0).reshape(BM, REL_DIM)


def _matmul(gp, gc, gs, gh, wp2, wc2, ws2, b2):
    pair_spec = pl.BlockSpec((BM2, PAIR), lambda i: (i, 0))
    w_spec = pl.BlockSpec((PAIR, 2 * REL_DIM), lambda i: (0, 0))
    return pl.pallas_call(
        _mm_body,
        grid=(HALF // BM2,),
        in_specs=[
            pair_spec, pair_spec, pair_spec,
            pl.BlockSpec((BM2, 2 * REL_DIM), lambda i: (i, 0)),
            w_spec, w_spec, w_spec,
            pl.BlockSpec((1, 2 * REL_DIM), lambda i: (0, 0)),
        ],
        out_specs=pl.BlockSpec((BM, REL_DIM), lambda i: (i, 0)),
        out_shape=jax.ShapeDtypeStruct((TOTAL, REL_DIM), jnp.float32),
    )(gp, gc, gs, gh, wp2, wc2, ws2, b2)


def _blockdiag(w):
    z = jnp.zeros((EMB_DIM, REL_DIM), jnp.float32)
    return jnp.concatenate(
        [jnp.concatenate([w, z], axis=1),
         jnp.concatenate([z, w], axis=1)], axis=0)


def kernel(input_data, index, src_enc_data, pos_table, cat_table, sense_table,
           W, b, lengths):
    ids = input_data.astype(jnp.int32)
    t = jnp.arange(TOTAL, dtype=jnp.int32)
    flat_idx = (t // SEQ_LEN) * SEQ_LEN + index.astype(jnp.int32)
    ipe, ice, ise = ids[0::2, 0], ids[0::2, 1], ids[0::2, 2]
    ipo, ico, iso = ids[1::2, 0], ids[1::2, 1], ids[1::2, 2]
    ihe, iho = flat_idx[0::2], flat_idx[1::2]
    wp = W[:EMB_DIM]
    wc = W[EMB_DIM:2 * EMB_DIM]
    ws = W[2 * EMB_DIM:3 * EMB_DIM]
    wh = W[3 * EMB_DIM:]
    hproj = _head_proj(src_enc_data, wh)
    gp, gc, gs = _emb_gather(ipe, ipo, ice, ico, ise, iso,
                             pos_table, cat_table, sense_table)
    gh = _head_gather(ihe, iho, hproj)
    return _matmul(gp, gc, gs, gh, _blockdiag(wp), _blockdiag(wc),
                   _blockdiag(ws),
                   jnp.concatenate([b, b]).reshape(1, 2 * REL_DIM))
